# baseline (speedup 1.0000x reference)
import functools

import jax
import jax.numpy as jnp
from jax import lax
from jax.experimental import pallas as pl
from jax.experimental.pallas import tpu as pltpu
from jax.experimental.pallas import tpu_sc as plsc

N_CARDS = 52
DIM = 64
BATCH = 16384
N_IDX = 200
W_WORDS = N_CARDS * DIM
NW = 32
BPW = BATCH // NW
LANES = 16

_mesh = plsc.VectorSubcoreMesh(core_axis_name="c", subcore_axis_name="s")


@functools.partial(
    pl.kernel,
    out_type=jax.ShapeDtypeStruct((N_IDX, DIM, BATCH), jnp.float32),
    mesh=_mesh,
    scratch_types=[
        pltpu.VMEM((W_WORDS,), jnp.float32),
        pltpu.VMEM((BPW,), jnp.int32),
        pltpu.VMEM((BPW,), jnp.int32),
        pltpu.VMEM((DIM, BPW), jnp.float32),
        pltpu.VMEM((DIM, BPW), jnp.float32),
        pltpu.SemaphoreType.DMA,
        pltpu.SemaphoreType.DMA,
        pltpu.SemaphoreType.DMA,
        pltpu.SemaphoreType.DMA,
    ],
    compiler_params=pltpu.CompilerParams(needs_layout_passes=False),
)
def _emb_lookup(idxt_hbm, w_hbm, out_hbm, w_v, idx0, idx1, rows0, rows1,
                si0, si1, ss0, ss1):
    wid = lax.axis_index("s") * 2 + lax.axis_index("c")
    b0 = wid * BPW
    idx_v = (idx0, idx1)
    rows_v = (rows0, rows1)
    sem_i = (si0, si1)
    sem_s = (ss0, ss1)

    pltpu.sync_copy(w_hbm, w_v)

    def idx_start(n, p):
        pltpu.async_copy(idxt_hbm.at[n, pl.ds(b0, BPW)], idx_v[p], sem_i[p])

    def idx_wait(n, p):
        pltpu.make_async_copy(idxt_hbm.at[n, pl.ds(b0, BPW)],
                              idx_v[p], sem_i[p]).wait()

    def scat_start(n, p):
        pltpu.async_copy(rows_v[p], out_hbm.at[n, :, pl.ds(b0, BPW)],
                         sem_s[p])

    def scat_wait(n, p):
        pltpu.make_async_copy(rows_v[p], out_hbm.at[n, :, pl.ds(b0, BPW)],
                              sem_s[p]).wait()

    def compute(p):
        iv = idx_v[p]
        rv = rows_v[p]

        def bg_body(bg, carry):
            for bt in range(BPW // 128):
                off = bt * 128 + bg * LANES
                idx16 = iv[pl.ds(off, LANES)]
                base = idx16 * DIM
                for d in range(DIM):
                    rv[d, pl.ds(off, LANES)] = plsc.load_gather(
                        w_v, [base + d])
            return carry

        lax.fori_loop(0, 128 // LANES, bg_body, 0)

    idx_start(0, 0)
    idx_start(1, 1)
    for p in (0, 1):
        idx_wait(p, p)
        compute(p)
        scat_start(p, p)
        idx_start(p + 2, p)

    def body(i, carry):
        for p in (0, 1):
            n = 2 * i + p
            idx_wait(n, p)
            scat_wait(n - 2, p)
            compute(p)
            scat_start(n, p)

            @pl.when(i < N_IDX // 2 - 1)
            def _():
                idx_start(n + 2, p)

        return carry

    lax.fori_loop(1, N_IDX // 2, body, 0)

    scat_wait(N_IDX - 2, 0)
    scat_wait(N_IDX - 1, 1)


def kernel(card_idxs, card_emb_weight):
    idxt = card_idxs.T
    w_flat = card_emb_weight.reshape(W_WORDS)
    out = _emb_lookup(idxt, w_flat)
    return out.transpose(2, 0, 1)


# speedup vs baseline: 7.6993x; 7.6993x over previous
import functools

import jax
import jax.numpy as jnp
from jax import lax
from jax.experimental import pallas as pl
from jax.experimental.pallas import tpu as pltpu
from jax.experimental.pallas import tpu_sc as plsc

N_CARDS = 52
DIM = 64
BATCH = 16384
N_IDX = 200
W_STRIDE = DIM + 1
W_WORDS = N_CARDS * W_STRIDE
NW = 32
BPW = BATCH // NW
LANES = 16

_mesh = plsc.VectorSubcoreMesh(core_axis_name="c", subcore_axis_name="s")


@functools.partial(
    pl.kernel,
    out_type=jax.ShapeDtypeStruct((N_IDX, DIM, BATCH), jnp.float32),
    mesh=_mesh,
    scratch_types=[
        pltpu.VMEM((W_WORDS,), jnp.float32),
        pltpu.VMEM((BPW,), jnp.int32),
        pltpu.VMEM((BPW,), jnp.int32),
        pltpu.VMEM((DIM, BPW), jnp.float32),
        pltpu.VMEM((DIM, BPW), jnp.float32),
        pltpu.SemaphoreType.DMA,
        pltpu.SemaphoreType.DMA,
        pltpu.SemaphoreType.DMA,
        pltpu.SemaphoreType.DMA,
    ],
    compiler_params=pltpu.CompilerParams(needs_layout_passes=False),
)
def _emb_lookup(idxt_hbm, w_hbm, out_hbm, w_v, idx0, idx1, rows0, rows1,
                si0, si1, ss0, ss1):
    wid = lax.axis_index("s") * 2 + lax.axis_index("c")
    b0 = wid * BPW
    idx_v = (idx0, idx1)
    rows_v = (rows0, rows1)
    sem_i = (si0, si1)
    sem_s = (ss0, ss1)

    pltpu.sync_copy(w_hbm, w_v)

    def idx_start(n, p):
        pltpu.async_copy(idxt_hbm.at[n, pl.ds(b0, BPW)], idx_v[p], sem_i[p])

    def idx_wait(n, p):
        pltpu.make_async_copy(idxt_hbm.at[n, pl.ds(b0, BPW)],
                              idx_v[p], sem_i[p]).wait()

    def scat_start(n, p):
        pltpu.async_copy(rows_v[p], out_hbm.at[n, :, pl.ds(b0, BPW)],
                         sem_s[p])

    def scat_wait(n, p):
        pltpu.make_async_copy(rows_v[p], out_hbm.at[n, :, pl.ds(b0, BPW)],
                              sem_s[p]).wait()

    def compute(p):
        iv = idx_v[p]
        rv = rows_v[p]

        def bg_body(bg, carry):
            for bt in range(BPW // 128):
                off = bt * 128 + bg * LANES
                idx16 = iv[pl.ds(off, LANES)]
                base = idx16 * W_STRIDE
                for d0 in range(0, DIM, 4):
                    g = [plsc.load_gather(w_v, [base + (d0 + k)])
                         for k in range(4)]
                    for k in range(4):
                        rv[d0 + k, pl.ds(off, LANES)] = g[k]
            return carry

        lax.fori_loop(0, 128 // LANES, bg_body, 0)

    idx_start(0, 0)
    idx_start(1, 1)
    for p in (0, 1):
        idx_wait(p, p)
        compute(p)
        scat_start(p, p)
        idx_start(p + 2, p)

    def body(i, carry):
        for p in (0, 1):
            n = 2 * i + p
            idx_wait(n, p)
            scat_wait(n - 2, p)
            compute(p)
            scat_start(n, p)

            @pl.when(i < N_IDX // 2 - 1)
            def _():
                idx_start(n + 2, p)

        return carry

    lax.fori_loop(1, N_IDX // 2, body, 0)

    scat_wait(N_IDX - 2, 0)
    scat_wait(N_IDX - 1, 1)


def kernel(card_idxs, card_emb_weight):
    idxt = card_idxs.T
    w_flat = jnp.pad(card_emb_weight,
                     ((0, 0), (0, W_STRIDE - DIM))).reshape(W_WORDS)
    out = _emb_lookup(idxt, w_flat)
    return out.transpose(2, 0, 1)
